# Initial kernel scaffold; baseline (speedup 1.0000x reference)
import jax
import jax.numpy as jnp
from jax.experimental import pallas as pl


def kernel(labels):
    raise NotImplementedError("write your pallas kernel here")


# speedup vs baseline: 1.2781x; 1.2781x over previous
import jax
import jax.numpy as jnp
from jax.experimental import pallas as pl


def _onehot_block(lab_ref, out_ref):
    blk, v = out_ref.shape
    iota = jax.lax.broadcasted_iota(jnp.int32, (blk, v), 1)
    out_ref[...] = (iota == lab_ref[...]).astype(jnp.float32)


def kernel(labels):
    if labels.ndim == 1:
        labels = labels.reshape(labels.shape[0], -1)
    b, f = labels.shape
    v = 1000
    rows = b * f
    blk = 2048
    while rows % blk != 0:
        blk //= 2
    flat = labels.reshape(rows, 1)
    out = pl.pallas_call(
        _onehot_block,
        grid=(rows // blk,),
        in_specs=[pl.BlockSpec((blk, 1), lambda i: (i, 0))],
        out_specs=pl.BlockSpec((blk, v), lambda i: (i, 0)),
        out_shape=jax.ShapeDtypeStruct((rows, v), jnp.float32),
    )(flat)
    return out.reshape(b, f * v)


# speedup vs baseline: 2.1019x; 1.6445x over previous
import jax
import jax.numpy as jnp
from jax.experimental import pallas as pl

_V = 1000


def _onehot_block(lab_ref, out_ref):
    blk, f = lab_ref.shape
    iota = jax.lax.broadcasted_iota(jnp.int32, (blk, _V), 1)
    for j in range(f):
        lab = lab_ref[:, j : j + 1]
        out_ref[:, j * _V : (j + 1) * _V] = (iota == lab).astype(jnp.float32)


def kernel(labels):
    if labels.ndim == 1:
        labels = labels.reshape(labels.shape[0], -1)
    b, f = labels.shape
    blk = 128
    while b % blk != 0:
        blk //= 2
    return pl.pallas_call(
        _onehot_block,
        grid=(b // blk,),
        in_specs=[pl.BlockSpec((blk, f), lambda i: (i, 0))],
        out_specs=pl.BlockSpec((blk, f * _V), lambda i: (i, 0)),
        out_shape=jax.ShapeDtypeStruct((b, f * _V), jnp.float32),
    )(labels)
